# Initial kernel scaffold; baseline (speedup 1.0000x reference)
import jax
import jax.numpy as jnp
from jax.experimental import pallas as pl


def kernel(x, edge_index, batch, W1, b1, W2, b2, W3, b3, Wlin, blin):
    raise NotImplementedError("write your pallas kernel here")


# speedup vs baseline: 15.1638x; 15.1638x over previous
import functools
import jax
import jax.numpy as jnp
from jax import lax
from jax.experimental import pallas as pl
from jax.experimental.pallas import tpu as pltpu
from jax.experimental.pallas import tpu_sc as plsc

N = 10000
E = 320000
D = 128
G = 64
C = 10
NC = 2
NT = 16
NW = NC * NT
EPW = E // NW
CH = 80
NCH = EPW // CH
RPT8 = 624
REM = N - NT * RPT8
DW = 16

BN = 1000
GRID = N // BN

def _zero_acc(zero_hbm, acc, t):
    base = pl.multiple_of(t * RPT8, 8)
    pltpu.sync_copy(zero_hbm.at[pl.ds(base, RPT8)], acc.at[pl.ds(base, RPT8)])

    @pl.when(t == NT - 1)
    def _():
        pltpu.sync_copy(zero_hbm.at[pl.ds(NT * RPT8, REM)],
                        acc.at[pl.ds(NT * RPT8, REM)])


def _writeback(acc, out_hbm, c, t):
    base = pl.multiple_of(t * RPT8, 8)
    pltpu.sync_copy(acc.at[pl.ds(base, RPT8)],
                    out_hbm.at[c, pl.ds(base, RPT8)])

    @pl.when(t == NT - 1)
    def _():
        pltpu.sync_copy(acc.at[pl.ds(NT * RPT8, REM)],
                        out_hbm.at[c, pl.ds(NT * RPT8, REM)])


def _sc_degree_body(dst3, ones_hbm, zero_hbm, out_hbm, acc, dst_v, ones_v):
    c = lax.axis_index("c")
    t = lax.axis_index("s")
    w = c * NT + t
    _zero_acc(zero_hbm, acc, t)
    pltpu.sync_copy(dst3.at[w], dst_v)
    pltpu.sync_copy(ones_hbm, ones_v)
    plsc.subcore_barrier()

    def chunk(j, carry):
        pltpu.sync_copy(ones_v, acc.at[dst_v.at[j]], add=True)
        return carry

    lax.fori_loop(0, NCH, chunk, 0)
    plsc.subcore_barrier()
    _writeback(acc, out_hbm, c, t)


def _sc_scatter_body(g_hbm, src3, dst3, zero_hbm, out_hbm,
                     acc, src_v, dst_v, rows_v, sem):
    c = lax.axis_index("c")
    t = lax.axis_index("s")
    w = c * NT + t
    _zero_acc(zero_hbm, acc, t)
    pltpu.sync_copy(src3.at[w], src_v)
    pltpu.sync_copy(dst3.at[w], dst_v)
    plsc.subcore_barrier()

    def chunk(j, carry):
        pltpu.async_copy(g_hbm.at[src_v.at[j]], rows_v, sem).wait()
        pltpu.sync_copy(rows_v, acc.at[dst_v.at[j]], add=True)
        return carry

    lax.fori_loop(0, NCH, chunk, 0)
    plsc.subcore_barrier()
    _writeback(acc, out_hbm, c, t)


@functools.lru_cache(maxsize=1)
def _sc_kernels():
    mesh = plsc.VectorSubcoreMesh(core_axis_name="c", subcore_axis_name="s")
    sc_degree = pl.kernel(
        _sc_degree_body,
        out_type=jax.ShapeDtypeStruct((NC, N, D), jnp.float32),
        mesh=mesh,
        scratch_types=[
            pltpu.VMEM_SHARED((N, D), jnp.float32),
            pltpu.VMEM((NCH, CH), jnp.int32),
            pltpu.VMEM((CH, D), jnp.float32),
        ],
    )
    sc_scatter = pl.kernel(
        _sc_scatter_body,
        out_type=jax.ShapeDtypeStruct((NC, N, D), jnp.float32),
        mesh=mesh,
        scratch_types=[
            pltpu.VMEM_SHARED((N, D), jnp.float32),
            pltpu.VMEM((NCH, CH), jnp.int32),
            pltpu.VMEM((NCH, CH), jnp.int32),
            pltpu.VMEM((CH, D), jnp.float32),
            pltpu.SemaphoreType.DMA,
        ],
    )
    return sc_degree, sc_scatter


def _dinv_of(dega, degb):
    deg = dega[:, :1] + degb[:, :1] + 1.0
    return lax.rsqrt(deg)


def _matT(h, w):
    return lax.dot_general(h, w, (((1,), (1,)), ((), ())),
                           preferred_element_type=jnp.float32)


def _tc1_body(x_ref, w1_ref, dega_ref, degb_ref, g_ref):
    dinv = _dinv_of(dega_ref[...], degb_ref[...])
    g_ref[...] = _matT(x_ref[...], w1_ref[...]) * dinv


def _tc2_body(sa_ref, sb_ref, gp_ref, dega_ref, degb_ref, b_ref, w_ref,
              g_ref):
    dinv = _dinv_of(dega_ref[...], degb_ref[...])
    h = jnp.maximum(
        dinv * (sa_ref[...] + sb_ref[...] + gp_ref[...]) + b_ref[...], 0.0)
    g_ref[...] = _matT(h, w_ref[...]) * dinv


def _tcf_body(sa_ref, sb_ref, gp_ref, dega_ref, degb_ref, b_ref, batch_ref,
              wlin_ref, blin_ref, out_ref, psum_ref, pcnt_ref):
    i = pl.program_id(0)

    @pl.when(i == 0)
    def _():
        psum_ref[...] = jnp.zeros_like(psum_ref)
        pcnt_ref[...] = jnp.zeros_like(pcnt_ref)

    dinv = _dinv_of(dega_ref[...], degb_ref[...])
    h = jnp.maximum(
        dinv * (sa_ref[...] + sb_ref[...] + gp_ref[...]) + b_ref[...], 0.0)
    bb = batch_ref[0, 0, :]
    m = (lax.broadcasted_iota(jnp.int32, (G, BN), 0)
         == bb[None, :]).astype(jnp.float32)
    psum_ref[...] += jnp.dot(m, h, preferred_element_type=jnp.float32)
    pcnt_ref[...] += jnp.dot(m, jnp.ones((BN, D), jnp.float32),
                             preferred_element_type=jnp.float32)

    @pl.when(i == pl.num_programs(0) - 1)
    def _():
        pooled = psum_ref[...] / jnp.maximum(pcnt_ref[...], 1.0)
        out_ref[...] = _matT(pooled, wlin_ref[...]) + blin_ref[...]


def _row_spec():
    return pl.BlockSpec((BN, D), lambda i: (i, 0))


def _deg_spec():
    return pl.BlockSpec((BN, DW), lambda i: (i, 0))


def _full_spec(shape):
    return pl.BlockSpec(shape, lambda i: tuple(0 for _ in shape))


_tc1 = pl.pallas_call(
    _tc1_body,
    grid=(GRID,),
    in_specs=[_row_spec(), _full_spec((D, D)), _deg_spec(), _deg_spec()],
    out_specs=_row_spec(),
    out_shape=jax.ShapeDtypeStruct((N, D), jnp.float32),
)

_tc2 = pl.pallas_call(
    _tc2_body,
    grid=(GRID,),
    in_specs=[_row_spec(), _row_spec(), _row_spec(), _deg_spec(), _deg_spec(),
              _full_spec((1, D)), _full_spec((D, D))],
    out_specs=_row_spec(),
    out_shape=jax.ShapeDtypeStruct((N, D), jnp.float32),
)

_tcf = pl.pallas_call(
    _tcf_body,
    grid=(GRID,),
    in_specs=[_row_spec(), _row_spec(), _row_spec(), _deg_spec(), _deg_spec(),
              _full_spec((1, D)),
              pl.BlockSpec((1, 1, BN), lambda i: (i, 0, 0)),
              _full_spec((C, D)), _full_spec((1, C))],
    out_specs=_full_spec((G, C)),
    out_shape=jax.ShapeDtypeStruct((G, C), jnp.float32),
    scratch_shapes=[pltpu.VMEM((G, D), jnp.float32),
                    pltpu.VMEM((G, D), jnp.float32)],
)


def kernel(x, edge_index, batch, W1, b1, W2, b2, W3, b3, Wlin, blin):
    src3 = edge_index[0].reshape(NW, NCH, CH)
    dst3 = edge_index[1].reshape(NW, NCH, CH)
    batch3 = batch.reshape(GRID, 1, BN)
    zeros = jnp.zeros((N, D), jnp.float32)
    ones_rows = jnp.ones((CH, D), jnp.float32)
    b1r = b1.reshape(1, D)
    b2r = b2.reshape(1, D)
    b3r = b3.reshape(1, D)
    blr = blin.reshape(1, C)

    _sc_degree, _sc_scatter = _sc_kernels()
    degp = _sc_degree(dst3, ones_rows, zeros)
    dega, degb = degp[0, :, :DW], degp[1, :, :DW]

    g1 = _tc1(x, W1, dega, degb)
    s1 = _sc_scatter(g1, src3, dst3, zeros)
    g2 = _tc2(s1[0], s1[1], g1, dega, degb, b1r, W2)
    s2 = _sc_scatter(g2, src3, dst3, zeros)
    g3 = _tc2(s2[0], s2[1], g2, dega, degb, b2r, W3)
    s3 = _sc_scatter(g3, src3, dst3, zeros)
    return _tcf(s3[0], s3[1], g3, dega, degb, b3r, batch3, Wlin, blr)


# speedup vs baseline: 19.3936x; 1.2789x over previous
import functools
import jax
import jax.numpy as jnp
from jax import lax
from jax.experimental import pallas as pl
from jax.experimental.pallas import tpu as pltpu
from jax.experimental.pallas import tpu_sc as plsc

N = 10000
E = 320000
D = 128
G = 64
C = 10
NC = 2
NT = 16
NW = NC * NT
EPW = E // NW
CH = 80
NCH = EPW // CH
RPT8 = 624
REM = N - NT * RPT8
DW = 16

BN = 1000
GRID = N // BN

def _zero_acc(zero_hbm, acc, t):
    base = pl.multiple_of(t * RPT8, 8)
    pltpu.sync_copy(zero_hbm.at[pl.ds(base, RPT8)], acc.at[pl.ds(base, RPT8)])

    @pl.when(t == NT - 1)
    def _():
        pltpu.sync_copy(zero_hbm.at[pl.ds(NT * RPT8, REM)],
                        acc.at[pl.ds(NT * RPT8, REM)])


def _writeback(acc, out_hbm, c, t):
    base = pl.multiple_of(t * RPT8, 8)
    pltpu.sync_copy(acc.at[pl.ds(base, RPT8)],
                    out_hbm.at[c, pl.ds(base, RPT8)])

    @pl.when(t == NT - 1)
    def _():
        pltpu.sync_copy(acc.at[pl.ds(NT * RPT8, REM)],
                        out_hbm.at[c, pl.ds(NT * RPT8, REM)])


def _sc_degree_body(dst3, ones_hbm, zero_hbm, out_hbm, acc, dst_v, ones_v):
    c = lax.axis_index("c")
    t = lax.axis_index("s")
    w = c * NT + t
    _zero_acc(zero_hbm, acc, t)
    pltpu.sync_copy(dst3.at[w], dst_v)
    pltpu.sync_copy(ones_hbm, ones_v)
    plsc.subcore_barrier()

    def chunk(j, carry):
        pltpu.sync_copy(ones_v, acc.at[dst_v.at[j]], add=True)
        return carry

    lax.fori_loop(0, NCH, chunk, 0)
    plsc.subcore_barrier()
    _writeback(acc, out_hbm, c, t)


def _sc_scatter_body(g_hbm, idx4, zero_hbm, out_hbm, acc,
                     is0, is1, rows0, rows1, ise0, ise1, gse0, gse1):
    c = lax.axis_index("c")
    t = lax.axis_index("s")
    w = c * NT + t
    s0 = (is0, ise0, rows0, gse0)
    s1 = (is1, ise1, rows1, gse1)

    def ifetch(j, s):
        pltpu.async_copy(idx4.at[w, j], s[0], s[1])

    def iwait(j, s):
        pltpu.make_async_copy(idx4.at[w, j], s[0], s[1]).wait()

    def gstart(s):
        pltpu.async_copy(g_hbm.at[s[0].at[0]], s[2], s[3])

    def gwait(s):
        pltpu.make_async_copy(g_hbm.at[s[0].at[0]], s[2], s[3]).wait()

    def scat(s):
        pltpu.sync_copy(s[2], acc.at[s[0].at[1]], add=True)

    _zero_acc(zero_hbm, acc, t)
    ifetch(0, s0)
    ifetch(1, s1)
    plsc.subcore_barrier()
    iwait(0, s0)
    gstart(s0)

    def pair(k, carry):
        jb = 2 * k + 2
        iwait(2 * k + 1, s1)
        gstart(s1)
        gwait(s0)
        scat(s0)
        ifetch(jb, s0)
        iwait(jb, s0)
        gstart(s0)
        gwait(s1)
        scat(s1)

        @pl.when(jb + 1 < NCH)
        def _():
            ifetch(jb + 1, s1)

        return carry

    lax.fori_loop(0, (NCH - 1) // 2, pair, 0)
    gwait(s0)
    scat(s0)
    plsc.subcore_barrier()
    _writeback(acc, out_hbm, c, t)


@functools.lru_cache(maxsize=1)
def _sc_kernels():
    mesh = plsc.VectorSubcoreMesh(core_axis_name="c", subcore_axis_name="s")
    sc_degree = pl.kernel(
        _sc_degree_body,
        out_type=jax.ShapeDtypeStruct((NC, N, D), jnp.float32),
        mesh=mesh,
        scratch_types=[
            pltpu.VMEM_SHARED((N, D), jnp.float32),
            pltpu.VMEM((NCH, CH), jnp.int32),
            pltpu.VMEM((CH, D), jnp.float32),
        ],
    )
    sc_scatter = pl.kernel(
        _sc_scatter_body,
        out_type=jax.ShapeDtypeStruct((NC, N, D), jnp.float32),
        mesh=mesh,
        scratch_types=[
            pltpu.VMEM_SHARED((N, D), jnp.float32),
            pltpu.VMEM((2, CH), jnp.int32),
            pltpu.VMEM((2, CH), jnp.int32),
            pltpu.VMEM((CH, D), jnp.float32),
            pltpu.VMEM((CH, D), jnp.float32),
            pltpu.SemaphoreType.DMA,
            pltpu.SemaphoreType.DMA,
            pltpu.SemaphoreType.DMA,
            pltpu.SemaphoreType.DMA,
        ],
    )
    return sc_degree, sc_scatter


def _dinv_of(dega, degb):
    deg = dega[:, :1] + degb[:, :1] + 1.0
    return lax.rsqrt(deg)


def _matT(h, w):
    return lax.dot_general(h, w, (((1,), (1,)), ((), ())),
                           preferred_element_type=jnp.float32)


def _tc1_body(x_ref, w1_ref, dega_ref, degb_ref, g_ref):
    dinv = _dinv_of(dega_ref[...], degb_ref[...])
    g_ref[...] = _matT(x_ref[...], w1_ref[...]) * dinv


def _tc2_body(sa_ref, sb_ref, gp_ref, dega_ref, degb_ref, b_ref, w_ref,
              g_ref):
    dinv = _dinv_of(dega_ref[...], degb_ref[...])
    h = jnp.maximum(
        dinv * (sa_ref[...] + sb_ref[...] + gp_ref[...]) + b_ref[...], 0.0)
    g_ref[...] = _matT(h, w_ref[...]) * dinv


def _tcf_body(sa_ref, sb_ref, gp_ref, dega_ref, degb_ref, b_ref, batch_ref,
              wlin_ref, blin_ref, out_ref, psum_ref, pcnt_ref):
    i = pl.program_id(0)

    @pl.when(i == 0)
    def _():
        psum_ref[...] = jnp.zeros_like(psum_ref)
        pcnt_ref[...] = jnp.zeros_like(pcnt_ref)

    dinv = _dinv_of(dega_ref[...], degb_ref[...])
    h = jnp.maximum(
        dinv * (sa_ref[...] + sb_ref[...] + gp_ref[...]) + b_ref[...], 0.0)
    bb = batch_ref[0, 0, :]
    m = (lax.broadcasted_iota(jnp.int32, (G, BN), 0)
         == bb[None, :]).astype(jnp.float32)
    psum_ref[...] += jnp.dot(m, h, preferred_element_type=jnp.float32)
    pcnt_ref[...] += jnp.dot(m, jnp.ones((BN, D), jnp.float32),
                             preferred_element_type=jnp.float32)

    @pl.when(i == pl.num_programs(0) - 1)
    def _():
        pooled = psum_ref[...] / jnp.maximum(pcnt_ref[...], 1.0)
        out_ref[...] = _matT(pooled, wlin_ref[...]) + blin_ref[...]


def _row_spec():
    return pl.BlockSpec((BN, D), lambda i: (i, 0))


def _deg_spec():
    return pl.BlockSpec((BN, DW), lambda i: (i, 0))


def _full_spec(shape):
    return pl.BlockSpec(shape, lambda i: tuple(0 for _ in shape))


_tc1 = pl.pallas_call(
    _tc1_body,
    grid=(GRID,),
    in_specs=[_row_spec(), _full_spec((D, D)), _deg_spec(), _deg_spec()],
    out_specs=_row_spec(),
    out_shape=jax.ShapeDtypeStruct((N, D), jnp.float32),
)

_tc2 = pl.pallas_call(
    _tc2_body,
    grid=(GRID,),
    in_specs=[_row_spec(), _row_spec(), _row_spec(), _deg_spec(), _deg_spec(),
              _full_spec((1, D)), _full_spec((D, D))],
    out_specs=_row_spec(),
    out_shape=jax.ShapeDtypeStruct((N, D), jnp.float32),
)

_tcf = pl.pallas_call(
    _tcf_body,
    grid=(GRID,),
    in_specs=[_row_spec(), _row_spec(), _row_spec(), _deg_spec(), _deg_spec(),
              _full_spec((1, D)),
              pl.BlockSpec((1, 1, BN), lambda i: (i, 0, 0)),
              _full_spec((C, D)), _full_spec((1, C))],
    out_specs=_full_spec((G, C)),
    out_shape=jax.ShapeDtypeStruct((G, C), jnp.float32),
    scratch_shapes=[pltpu.VMEM((G, D), jnp.float32),
                    pltpu.VMEM((G, D), jnp.float32)],
)


def kernel(x, edge_index, batch, W1, b1, W2, b2, W3, b3, Wlin, blin):
    src3 = edge_index[0].reshape(NW, NCH, CH)
    dst3 = edge_index[1].reshape(NW, NCH, CH)
    idx4 = jnp.stack([src3, dst3], axis=2)
    batch3 = batch.reshape(GRID, 1, BN)
    zeros = jnp.zeros((N, D), jnp.float32)
    ones_rows = jnp.ones((CH, D), jnp.float32)
    b1r = b1.reshape(1, D)
    b2r = b2.reshape(1, D)
    b3r = b3.reshape(1, D)
    blr = blin.reshape(1, C)

    _sc_degree, _sc_scatter = _sc_kernels()
    degp = _sc_degree(dst3, ones_rows, zeros)
    dega, degb = degp[0, :, :DW], degp[1, :, :DW]

    g1 = _tc1(x, W1, dega, degb)
    s1 = _sc_scatter(g1, idx4, zeros)
    g2 = _tc2(s1[0], s1[1], g1, dega, degb, b1r, W2)
    s2 = _sc_scatter(g2, idx4, zeros)
    g3 = _tc2(s2[0], s2[1], g2, dega, degb, b2r, W3)
    s3 = _sc_scatter(g3, idx4, zeros)
    return _tcf(s3[0], s3[1], g3, dega, degb, b3r, batch3, Wlin, blr)


# speedup vs baseline: 20.2313x; 1.0432x over previous
import functools
import jax
import jax.numpy as jnp
from jax import lax
from jax.experimental import pallas as pl
from jax.experimental.pallas import tpu as pltpu
from jax.experimental.pallas import tpu_sc as plsc

N = 10000
E = 320000
D = 128
G = 64
C = 10
NC = 2
NT = 16
NW = NC * NT
EPW = E // NW
CH = 80
NCH = EPW // CH
RPT8 = 624
REM = N - NT * RPT8
DW = 16

BN = 1000
GRID = N // BN

def _zero_acc(zero_hbm, acc, t):
    base = pl.multiple_of(t * RPT8, 8)
    pltpu.sync_copy(zero_hbm.at[pl.ds(base, RPT8)], acc.at[pl.ds(base, RPT8)])

    @pl.when(t == NT - 1)
    def _():
        pltpu.sync_copy(zero_hbm.at[pl.ds(NT * RPT8, REM)],
                        acc.at[pl.ds(NT * RPT8, REM)])


def _writeback(acc, out_hbm, c, t):
    base = pl.multiple_of(t * RPT8, 8)
    pltpu.sync_copy(acc.at[pl.ds(base, RPT8)],
                    out_hbm.at[c, pl.ds(base, RPT8)])

    @pl.when(t == NT - 1)
    def _():
        pltpu.sync_copy(acc.at[pl.ds(NT * RPT8, REM)],
                        out_hbm.at[c, pl.ds(NT * RPT8, REM)])


def _sc_degree_body(dst3, ones_hbm, zero_hbm, out_hbm, acc, dst_v, ones_v,
                    sem):
    c = lax.axis_index("c")
    t = lax.axis_index("s")
    w = c * NT + t
    _zero_acc(zero_hbm, acc, t)
    pltpu.sync_copy(dst3.at[w], dst_v)
    pltpu.sync_copy(ones_hbm, ones_v)
    plsc.subcore_barrier()

    def fire(j, carry):
        pltpu.async_copy(ones_v, acc.at[dst_v.at[j]], sem, add=True)
        return carry

    def drain(j, carry):
        pltpu.make_async_copy(ones_v, acc.at[dst_v.at[j]], sem).wait()
        return carry

    lax.fori_loop(0, NCH, fire, 0)
    lax.fori_loop(0, NCH, drain, 0)
    plsc.subcore_barrier()
    _writeback(acc, out_hbm, c, t)


def _sc_scatter_body(g_hbm, idx4, zero_hbm, out_hbm, acc,
                     is0, is1, rows0, rows1, ise0, ise1, gse0, gse1):
    c = lax.axis_index("c")
    t = lax.axis_index("s")
    w = c * NT + t
    s0 = (is0, ise0, rows0, gse0)
    s1 = (is1, ise1, rows1, gse1)

    def ifetch(j, s):
        pltpu.async_copy(idx4.at[w, j], s[0], s[1])

    def iwait(j, s):
        pltpu.make_async_copy(idx4.at[w, j], s[0], s[1]).wait()

    def gstart(s):
        pltpu.async_copy(g_hbm.at[s[0].at[0]], s[2], s[3])

    def gwait(s):
        pltpu.make_async_copy(g_hbm.at[s[0].at[0]], s[2], s[3]).wait()

    def scat(s):
        pltpu.sync_copy(s[2], acc.at[s[0].at[1]], add=True)

    _zero_acc(zero_hbm, acc, t)
    ifetch(0, s0)
    ifetch(1, s1)
    plsc.subcore_barrier()
    iwait(0, s0)
    gstart(s0)

    def pair(k, carry):
        jb = 2 * k + 2
        iwait(2 * k + 1, s1)
        gstart(s1)
        gwait(s0)
        scat(s0)
        ifetch(jb, s0)
        iwait(jb, s0)
        gstart(s0)
        gwait(s1)
        scat(s1)

        @pl.when(jb + 1 < NCH)
        def _():
            ifetch(jb + 1, s1)

        return carry

    lax.fori_loop(0, (NCH - 1) // 2, pair, 0)
    gwait(s0)
    scat(s0)
    plsc.subcore_barrier()
    _writeback(acc, out_hbm, c, t)


@functools.lru_cache(maxsize=1)
def _sc_kernels():
    mesh = plsc.VectorSubcoreMesh(core_axis_name="c", subcore_axis_name="s")
    sc_degree = pl.kernel(
        _sc_degree_body,
        out_type=jax.ShapeDtypeStruct((NC, N, DW), jnp.float32),
        mesh=mesh,
        scratch_types=[
            pltpu.VMEM_SHARED((N, DW), jnp.float32),
            pltpu.VMEM((NCH, CH), jnp.int32),
            pltpu.VMEM((CH, DW), jnp.float32),
            pltpu.SemaphoreType.DMA,
        ],
    )
    sc_scatter = pl.kernel(
        _sc_scatter_body,
        out_type=jax.ShapeDtypeStruct((NC, N, D), jnp.float32),
        mesh=mesh,
        scratch_types=[
            pltpu.VMEM_SHARED((N, D), jnp.float32),
            pltpu.VMEM((2, CH), jnp.int32),
            pltpu.VMEM((2, CH), jnp.int32),
            pltpu.VMEM((CH, D), jnp.float32),
            pltpu.VMEM((CH, D), jnp.float32),
            pltpu.SemaphoreType.DMA,
            pltpu.SemaphoreType.DMA,
            pltpu.SemaphoreType.DMA,
            pltpu.SemaphoreType.DMA,
        ],
    )
    return sc_degree, sc_scatter


def _dinv_of(dega, degb):
    deg = dega[:, :1] + degb[:, :1] + 1.0
    return lax.rsqrt(deg)


def _matT(h, w):
    return lax.dot_general(h, w, (((1,), (1,)), ((), ())),
                           preferred_element_type=jnp.float32)


def _tc1_body(x_ref, w1_ref, dega_ref, degb_ref, g_ref):
    dinv = _dinv_of(dega_ref[...], degb_ref[...])
    g_ref[...] = _matT(x_ref[...], w1_ref[...]) * dinv


def _tc2_body(sa_ref, sb_ref, gp_ref, dega_ref, degb_ref, b_ref, w_ref,
              g_ref):
    dinv = _dinv_of(dega_ref[...], degb_ref[...])
    h = jnp.maximum(
        dinv * (sa_ref[...] + sb_ref[...] + gp_ref[...]) + b_ref[...], 0.0)
    g_ref[...] = _matT(h, w_ref[...]) * dinv


def _tcf_body(sa_ref, sb_ref, gp_ref, dega_ref, degb_ref, b_ref, batch_ref,
              wlin_ref, blin_ref, out_ref, psum_ref, pcnt_ref):
    i = pl.program_id(0)

    @pl.when(i == 0)
    def _():
        psum_ref[...] = jnp.zeros_like(psum_ref)
        pcnt_ref[...] = jnp.zeros_like(pcnt_ref)

    dinv = _dinv_of(dega_ref[...], degb_ref[...])
    h = jnp.maximum(
        dinv * (sa_ref[...] + sb_ref[...] + gp_ref[...]) + b_ref[...], 0.0)
    bb = batch_ref[0, 0, :]
    m = (lax.broadcasted_iota(jnp.int32, (G, BN), 0)
         == bb[None, :]).astype(jnp.float32)
    psum_ref[...] += jnp.dot(m, h, preferred_element_type=jnp.float32)
    pcnt_ref[...] += jnp.dot(m, jnp.ones((BN, D), jnp.float32),
                             preferred_element_type=jnp.float32)

    @pl.when(i == pl.num_programs(0) - 1)
    def _():
        pooled = psum_ref[...] / jnp.maximum(pcnt_ref[...], 1.0)
        out_ref[...] = _matT(pooled, wlin_ref[...]) + blin_ref[...]


def _row_spec():
    return pl.BlockSpec((BN, D), lambda i: (i, 0))


def _deg_spec():
    return pl.BlockSpec((BN, DW), lambda i: (i, 0))


def _full_spec(shape):
    return pl.BlockSpec(shape, lambda i: tuple(0 for _ in shape))


_tc1 = pl.pallas_call(
    _tc1_body,
    grid=(GRID,),
    in_specs=[_row_spec(), _full_spec((D, D)), _deg_spec(), _deg_spec()],
    out_specs=_row_spec(),
    out_shape=jax.ShapeDtypeStruct((N, D), jnp.float32),
)

_tc2 = pl.pallas_call(
    _tc2_body,
    grid=(GRID,),
    in_specs=[_row_spec(), _row_spec(), _row_spec(), _deg_spec(), _deg_spec(),
              _full_spec((1, D)), _full_spec((D, D))],
    out_specs=_row_spec(),
    out_shape=jax.ShapeDtypeStruct((N, D), jnp.float32),
)

_tcf = pl.pallas_call(
    _tcf_body,
    grid=(GRID,),
    in_specs=[_row_spec(), _row_spec(), _row_spec(), _deg_spec(), _deg_spec(),
              _full_spec((1, D)),
              pl.BlockSpec((1, 1, BN), lambda i: (i, 0, 0)),
              _full_spec((C, D)), _full_spec((1, C))],
    out_specs=_full_spec((G, C)),
    out_shape=jax.ShapeDtypeStruct((G, C), jnp.float32),
    scratch_shapes=[pltpu.VMEM((G, D), jnp.float32),
                    pltpu.VMEM((G, D), jnp.float32)],
)


def kernel(x, edge_index, batch, W1, b1, W2, b2, W3, b3, Wlin, blin):
    src3 = edge_index[0].reshape(NW, NCH, CH)
    dst3 = edge_index[1].reshape(NW, NCH, CH)
    idx4 = jnp.stack([src3, dst3], axis=2)
    batch3 = batch.reshape(GRID, 1, BN)
    zeros = jnp.zeros((N, D), jnp.float32)
    zeros_deg = jnp.zeros((N, DW), jnp.float32)
    ones_rows = jnp.ones((CH, DW), jnp.float32)
    b1r = b1.reshape(1, D)
    b2r = b2.reshape(1, D)
    b3r = b3.reshape(1, D)
    blr = blin.reshape(1, C)

    _sc_degree, _sc_scatter = _sc_kernels()
    degp = _sc_degree(dst3, ones_rows, zeros_deg)
    dega, degb = degp[0], degp[1]

    g1 = _tc1(x, W1, dega, degb)
    s1 = _sc_scatter(g1, idx4, zeros)
    g2 = _tc2(s1[0], s1[1], g1, dega, degb, b1r, W2)
    s2 = _sc_scatter(g2, idx4, zeros)
    g3 = _tc2(s2[0], s2[1], g2, dega, degb, b2r, W3)
    s3 = _sc_scatter(g3, idx4, zeros)
    return _tcf(s3[0], s3[1], g3, dega, degb, b3r, batch3, Wlin, blr)
